# baseline (speedup 1.0000x reference)
import functools

import jax
import jax.numpy as jnp
from jax import lax
from jax.experimental import pallas as pl
from jax.experimental.pallas import tpu as pltpu
from jax.experimental.pallas import tpu_sc as plsc

N = 10000
E = 320000
EMB = 128
NUM_LAYER = 5
NUM_GRAPHS = 64
NUM_CLASS = 10
BN_EPS = 1e-5

NC = 2
NS = 16
NW = NC * NS
CHUNK = 64
KB2 = 2 * (-(-E // (NW * CHUNK)))
KB0 = 250
KB1 = KB2 - KB0
TOTCH = NS * KB2
EP = TOTCH * CHUNK
ACC_ROWS = 10240
RPT = ACC_ROWS // NS
DUMMY_ROW = N + 8

BLK = 1000
GRID = N // BLK


NBUF = 4
DG = NBUF - 1
DI = 6
NIDX = 8


def _sc_agg_body(hl, eip, out, idx_v, rows_v, acc, sem_i, sem_g,
                 sem_s):
    c = lax.axis_index("c")
    s = lax.axis_index("s")
    kb = jnp.where(c == 0, KB0, KB1)
    base = jnp.where(c == 0, s * KB0, NS * KB0 + s * KB1)

    zv = jnp.zeros((16,), jnp.float32)

    def zstore(k, carry):
        r = lax.shift_right_logical(k, 3)
        q = lax.bitwise_and(k, 7)
        rows_v[0, r, pl.ds(q * 16, 16)] = zv
        return carry

    lax.fori_loop(0, CHUNK * (EMB // 16), zstore, 0, unroll=False)
    for t in range(RPT // CHUNK):
        pltpu.sync_copy(rows_v.at[0], acc.at[pl.ds(s * RPT + t * CHUNK,
                                                   CHUNK)])
    plsc.subcore_barrier()

    def fire_idx(j):
        sl = lax.rem(j, NIDX)
        pltpu.async_copy(eip.at[base + j], idx_v.at[sl], sem_i.at[sl])

    def wait_idx(j):
        sl = lax.rem(j, NIDX)
        pltpu.make_async_copy(eip.at[base + j], idx_v.at[sl],
                              sem_i.at[sl]).wait()

    def gather(j, b):
        sl = lax.rem(j, NIDX)
        pltpu.async_copy(hl.at[idx_v.at[sl, 0]], rows_v.at[b], sem_g.at[b])

    def wait_g(j, b):
        sl = lax.rem(j, NIDX)
        pltpu.make_async_copy(hl.at[idx_v.at[sl, 0]], rows_v.at[b],
                              sem_g.at[b]).wait()

    def scat(j, b):
        sl = lax.rem(j, NIDX)
        pltpu.async_copy(rows_v.at[b], acc.at[idx_v.at[sl, 1]], sem_s.at[b],
                         add=True)

    def wait_s(j, b):
        sl = lax.rem(j, NIDX)
        pltpu.make_async_copy(rows_v.at[b], acc.at[idx_v.at[sl, 1]],
                              sem_s.at[b]).wait()

    for k in range(DI):
        fire_idx(k)
    for k in range(DG):
        wait_idx(k)
        gather(k, k)

    wait_g(0, 0)
    scat(0, 0)
    wait_idx(DG)
    gather(DG, DG)
    fire_idx(DI)

    def body(fire, j, carry):
        bp = lax.rem(j - 1, NBUF)
        wait_s(j - 1, bp)
        wait_idx(j + DG)
        gather(j + DG, bp)
        if fire:
            fire_idx(j + DI)
        b = lax.rem(j, NBUF)
        wait_g(j, b)
        scat(j, b)
        return carry

    lax.fori_loop(1, kb - DI, functools.partial(body, True), 0, unroll=False)
    lax.fori_loop(kb - DI, kb - DG, functools.partial(body, False), 0,
                  unroll=False)

    for dj in range(DG):
        j = kb - DG + dj
        b = lax.rem(j, NBUF)
        wait_g(j, b)
        scat(j, b)
    for dt in range(DG + 1):
        t = kb - DG - 1 + dt
        wait_s(t, lax.rem(t, NBUF))

    plsc.subcore_barrier()
    pltpu.sync_copy(acc.at[pl.ds(s * RPT, RPT)], out.at[c, pl.ds(s * RPT, RPT)])


NI2 = 2


DPAD = 1024


DROWS = (GRID + 1) * DPAD
DSLC = DPAD


def _sc_deg_body(eip, zeros1, out, dbuf, tidx, ones_v, acc1, sem_i, sem_s):
    c = lax.axis_index("c")
    s = lax.axis_index("s")
    kb = jnp.where(c == 0, KB0, KB1)
    base = jnp.where(c == 0, s * KB0, NS * KB0 + s * KB1)

    @pl.when(s < GRID + 1)
    def _():
        pltpu.sync_copy(zeros1.at[s], acc1.at[pl.ds(s * DPAD, DPAD)])
    for q in range(CHUNK // 16):
        ones_v[pl.ds(q * 16, 16)] = jnp.ones((16,), jnp.float32)
    plsc.subcore_barrier()

    def fire(j):
        sl = lax.rem(j, NI2)
        pltpu.async_copy(eip.at[base + j], dbuf.at[sl], sem_i.at[sl])

    def wait(j):
        sl = lax.rem(j, NI2)
        pltpu.make_async_copy(eip.at[base + j], dbuf.at[sl],
                              sem_i.at[sl]).wait()

    def scat(j):
        sl = lax.rem(j, NI2)
        pltpu.async_copy(ones_v, acc1.at[tidx.at[sl]], sem_s.at[sl],
                         add=True)

    def wait_s(j):
        sl = lax.rem(j, NI2)
        pltpu.make_async_copy(ones_v, acc1.at[tidx.at[sl]],
                              sem_s.at[sl]).wait()

    def xform(j):
        sl = lax.rem(j, NI2)
        for q in range(CHUNK // 16):
            iv = dbuf[sl, 1, pl.ds(q * 16, 16)]
            gv = lax.shift_right_logical(iv * 33555, 25)
            fi = iv - gv * BLK + lax.shift_left(gv, 10)
            tidx[sl, pl.ds(q * 16, 16)] = fi

    fire(0)
    fire(1)
    wait(0)
    xform(0)
    fire(2)
    scat(0)
    wait(1)
    xform(1)
    fire(3)
    scat(1)

    def body(fires, j, carry):
        wait(j)
        wait_s(j - 2)
        xform(j)
        if fires:
            fire(j + NI2)
        scat(j)
        return carry

    lax.fori_loop(2, kb - NI2, functools.partial(body, True), 0,
                  unroll=False)
    lax.fori_loop(kb - NI2, kb, functools.partial(body, False), 0,
                  unroll=False)

    wait_s(kb - 2)
    wait_s(kb - 1)
    plsc.subcore_barrier()

    @pl.when(s < GRID)
    def _():
        pltpu.sync_copy(acc1.at[pl.ds(s * DPAD, DPAD)], out.at[s, c])


@functools.partial(jax.jit, static_argnames=())
def _sc_deg(eip, zeros1):
    mesh = plsc.VectorSubcoreMesh(core_axis_name="c", subcore_axis_name="s")
    f = pl.kernel(
        _sc_deg_body,
        out_type=jax.ShapeDtypeStruct((GRID, NC, DPAD), jnp.float32),
        mesh=mesh,
        scratch_types=[
            pltpu.VMEM((NI2, 2, CHUNK), jnp.int32),
            pltpu.VMEM((NI2, CHUNK), jnp.int32),
            pltpu.VMEM((CHUNK,), jnp.float32),
            pltpu.VMEM_SHARED((DROWS,), jnp.float32),
            pltpu.SemaphoreType.DMA((NI2,)),
            pltpu.SemaphoreType.DMA((NI2,)),
        ],
    )
    return f(eip, zeros1)


@functools.partial(jax.jit, static_argnames=())
def _sc_agg(hl, eip):
    mesh = plsc.VectorSubcoreMesh(core_axis_name="c", subcore_axis_name="s")
    f = pl.kernel(
        _sc_agg_body,
        out_type=jax.ShapeDtypeStruct((NC, ACC_ROWS, EMB), jnp.float32),
        mesh=mesh,
        scratch_types=[
            pltpu.VMEM((NIDX, 2, CHUNK), jnp.int32),
            pltpu.VMEM((NBUF, CHUNK, EMB), jnp.float32),
            pltpu.VMEM_SHARED((ACC_ROWS, EMB), jnp.float32),
            pltpu.SemaphoreType.DMA((NIDX,)),
            pltpu.SemaphoreType.DMA((NBUF,)),
            pltpu.SemaphoreType.DMA((NBUF,)),
        ],
    )
    return f(hl, eip)


def _one_hot_t(b3_blk):
    bt = b3_blk.reshape(1, BLK)
    gi = lax.broadcasted_iota(jnp.int32, (NUM_GRAPHS, BLK), 0)
    return (jnp.broadcast_to(bt, (NUM_GRAPHS, BLK)) == gi).astype(jnp.float32)


def _mlp_tail(i, last, z, b3_ref, w1_ref, b1_ref, w2_ref, b2_ref, vn_ref,
              hln_ref, pool_ref):
    y = jnp.dot(z, w1_ref[...], preferred_element_type=jnp.float32) + b1_ref[...]
    y = jnp.maximum(y, 0.0)
    h = jnp.dot(y, w2_ref[...], preferred_element_type=jnp.float32) + b2_ref[...]
    if not last:
        h = jnp.maximum(h, 0.0)
    bT = _one_hot_t(b3_ref[...])
    vnb = lax.dot_general(bT, vn_ref[...], (((0,), (0,)), ((), ())),
                          preferred_element_type=jnp.float32)
    hln = h + vnb
    hln_ref[...] = hln
    p = jnp.dot(bT, hln, preferred_element_type=jnp.float32)

    @pl.when(i == 0)
    def _():
        pool_ref[...] = p

    @pl.when(i > 0)
    def _():
        pool_ref[...] += p


def _tc_layer_body(last, hl_ref, acc_ref, b3_ref, w1_ref, b1_ref, w2_ref,
                   b2_ref, epsr_ref, vn_ref, hln_ref, pool_ref):
    i = pl.program_id(0)
    acc = acc_ref[...]
    z = hl_ref[...] * epsr_ref[...] + acc[0] + acc[1]
    _mlp_tail(i, last, z, b3_ref, w1_ref, b1_ref, w2_ref, b2_ref, vn_ref,
              hln_ref, pool_ref)


def _tc_layer0_body(deg_ref, b3_ref, c0_ref, epsr_ref, w1_ref, b1_ref,
                    w2_ref, b2_ref, vn_ref, hln_ref, pool_ref):
    i = pl.program_id(0)
    degT = jnp.sum(deg_ref[...].reshape(NC, DPAD), axis=0,
                   keepdims=True)[:, :BLK]
    c0 = c0_ref[...]
    z = (jnp.broadcast_to(c0 * epsr_ref[...], (BLK, EMB))
         + lax.dot_general(degT, c0, (((0,), (0,)), ((), ())),
                           preferred_element_type=jnp.float32))
    _mlp_tail(i, False, z, b3_ref, w1_ref, b1_ref, w2_ref, b2_ref, vn_ref,
              hln_ref, pool_ref)


def _tc_layer0(deg_p, b3, c0row, epsr, w1f, b1f, w2f, b2f, vn_next):
    return pl.pallas_call(
        _tc_layer0_body,
        grid=(GRID,),
        in_specs=[
            pl.BlockSpec((1, NC, DPAD), lambda i: (i, 0, 0)),
            pl.BlockSpec((1, 1, BLK), lambda i: (i, 0, 0)),
            pl.BlockSpec((1, EMB), lambda i: (0, 0)),
            pl.BlockSpec((1, EMB), lambda i: (0, 0)),
            pl.BlockSpec((EMB, 2 * EMB), lambda i: (0, 0)),
            pl.BlockSpec((1, 2 * EMB), lambda i: (0, 0)),
            pl.BlockSpec((2 * EMB, EMB), lambda i: (0, 0)),
            pl.BlockSpec((1, EMB), lambda i: (0, 0)),
            pl.BlockSpec((NUM_GRAPHS, EMB), lambda i: (0, 0)),
        ],
        out_specs=[
            pl.BlockSpec((BLK, EMB), lambda i: (i, 0)),
            pl.BlockSpec((NUM_GRAPHS, EMB), lambda i: (0, 0)),
        ],
        out_shape=[
            jax.ShapeDtypeStruct((N, EMB), jnp.float32),
            jax.ShapeDtypeStruct((NUM_GRAPHS, EMB), jnp.float32),
        ],
        compiler_params=pltpu.CompilerParams(
            dimension_semantics=("arbitrary",)),
    )(deg_p, b3, c0row, epsr, w1f, b1f, w2f, b2f, vn_next)


def _tc_layer(hl, acc, b3, w1f, b1f, w2f, b2f, epsr, vn_next, last):
    return pl.pallas_call(
        functools.partial(_tc_layer_body, last),
        grid=(GRID,),
        in_specs=[
            pl.BlockSpec((BLK, EMB), lambda i: (i, 0)),
            pl.BlockSpec((NC, BLK, EMB), lambda i: (0, i, 0)),
            pl.BlockSpec((1, 1, BLK), lambda i: (i, 0, 0)),
            pl.BlockSpec((EMB, 2 * EMB), lambda i: (0, 0)),
            pl.BlockSpec((1, 2 * EMB), lambda i: (0, 0)),
            pl.BlockSpec((2 * EMB, EMB), lambda i: (0, 0)),
            pl.BlockSpec((1, EMB), lambda i: (0, 0)),
            pl.BlockSpec((1, EMB), lambda i: (0, 0)),
            pl.BlockSpec((NUM_GRAPHS, EMB), lambda i: (0, 0)),
        ],
        out_specs=[
            pl.BlockSpec((BLK, EMB), lambda i: (i, 0)),
            pl.BlockSpec((NUM_GRAPHS, EMB), lambda i: (0, 0)),
        ],
        out_shape=[
            jax.ShapeDtypeStruct((N, EMB), jnp.float32),
            jax.ShapeDtypeStruct((NUM_GRAPHS, EMB), jnp.float32),
        ],
        compiler_params=pltpu.CompilerParams(
            dimension_semantics=("arbitrary",)),
    )(hl, acc, b3, w1f, b1f, w2f, b2f, epsr, vn_next)


def _tc_cnt_body(b3_ref, cnt_ref):
    i = pl.program_id(0)
    bT = _one_hot_t(b3_ref[...])
    c = jnp.broadcast_to(jnp.sum(bT, axis=1, keepdims=True),
                         (NUM_GRAPHS, EMB))

    @pl.when(i == 0)
    def _():
        cnt_ref[...] = c

    @pl.when(i > 0)
    def _():
        cnt_ref[...] += c


def _tc_cnt(b3):
    return pl.pallas_call(
        _tc_cnt_body,
        grid=(GRID,),
        in_specs=[pl.BlockSpec((1, 1, BLK), lambda i: (i, 0, 0))],
        out_specs=pl.BlockSpec((NUM_GRAPHS, EMB), lambda i: (0, 0)),
        out_shape=jax.ShapeDtypeStruct((NUM_GRAPHS, EMB), jnp.float32),
        compiler_params=pltpu.CompilerParams(
            dimension_semantics=("arbitrary",)),
    )(b3)


def _tc_vn0_body(cnt_ref, c0_ref, vn_ref, q1_ref, qb1_ref, q2_ref, qb2_ref,
                 out_ref):
    vt = (cnt_ref[...] * jnp.broadcast_to(c0_ref[...],
                                          (NUM_GRAPHS, EMB))
          + vn_ref[...])
    v = jnp.dot(vt, q1_ref[...], preferred_element_type=jnp.float32) + qb1_ref[...]
    v = jnp.maximum(v, 0.0)
    v = jnp.dot(v, q2_ref[...], preferred_element_type=jnp.float32) + qb2_ref[...]
    out_ref[...] = jnp.maximum(v, 0.0)


def _tc_vn0(cnt, c0row, vn, q1, qb1, q2, qb2):
    return pl.pallas_call(
        _tc_vn0_body,
        out_shape=jax.ShapeDtypeStruct((NUM_GRAPHS, EMB), jnp.float32),
    )(cnt, c0row, vn, q1, qb1, q2, qb2)


def _tc_vn_body(pool_ref, vn_ref, q1_ref, qb1_ref, q2_ref, qb2_ref, out_ref):
    vt = pool_ref[...] + vn_ref[...]
    v = jnp.dot(vt, q1_ref[...], preferred_element_type=jnp.float32) + qb1_ref[...]
    v = jnp.maximum(v, 0.0)
    v = jnp.dot(v, q2_ref[...], preferred_element_type=jnp.float32) + qb2_ref[...]
    out_ref[...] = jnp.maximum(v, 0.0)


def _tc_vn(pool, vn, q1, qb1, q2, qb2):
    return pl.pallas_call(
        _tc_vn_body,
        out_shape=jax.ShapeDtypeStruct((NUM_GRAPHS, EMB), jnp.float32),
    )(pool, vn, q1, qb1, q2, qb2)


def _tc_final_body(pool_ref, cnt_ref, w_ref, b_ref, out_ref):
    hg = pool_ref[...] / jnp.maximum(cnt_ref[...], 1.0)
    out_ref[...] = (jnp.dot(hg, w_ref[...], preferred_element_type=jnp.float32)
                    + b_ref[...])


def _tc_final(pool, cnt, w, b):
    return pl.pallas_call(
        _tc_final_body,
        out_shape=jax.ShapeDtypeStruct((NUM_GRAPHS, NUM_CLASS), jnp.float32),
    )(pool, cnt, w, b)


_BN_S = (1.0 + BN_EPS) ** -0.5


def _fold(W1, b1, g1, bb1, W2, b2, g2, bb2):
    s1 = _BN_S * g1
    s2 = _BN_S * g2
    return (W1 * s1[None, :], (b1 * s1 + bb1)[None, :],
            W2 * s2[None, :], (b2 * s2 + bb2)[None, :])


def kernel(x, edge_index, batch, params):
    del x
    src = edge_index[0]
    dst = edge_index[1]
    pad = EP - E
    srcp = jnp.concatenate([src, jnp.zeros((pad,), jnp.int32)]).reshape(
        TOTCH, 1, CHUNK)
    dstp = jnp.concatenate(
        [dst, jnp.full((pad,), DUMMY_ROW, jnp.int32)]).reshape(TOTCH, 1,
                                                               CHUNK)
    eip = jnp.concatenate([srcp, dstp], axis=1)
    b3 = batch.reshape(GRID, 1, BLK)

    c0row = (params['atom_emb'][0] + params['vn_emb'][0])[None, :]
    vn = jnp.broadcast_to(params['vn_emb'][0], (NUM_GRAPHS, EMB))

    zeros1 = jnp.zeros((GRID + 1, DPAD), jnp.float32)
    counts = _tc_cnt(b3)
    deg_p = _sc_deg(eip, zeros1)

    p = params['layers'][0]
    w1f, b1f, w2f, b2f = _fold(p['W1'], p['b1'], p['bn1_g'], p['bn1_b'],
                               p['W2'], p['b2'], p['bn_g'], p['bn_b'])
    epsr = jnp.broadcast_to(1.0 + p['eps'], (1, EMB)).astype(jnp.float32)
    q = params['vn_mlps'][0]
    q1f, qb1f, q2f, qb2f = _fold(q['W1'], q['b1'], q['bn1_g'], q['bn1_b'],
                                 q['W2'], q['b2'], q['bn2_g'], q['bn2_b'])
    vn = _tc_vn0(counts, c0row, vn, q1f, qb1f, q2f, qb2f)
    hl, pool = _tc_layer0(deg_p, b3, c0row, epsr, w1f, b1f, w2f, b2f, vn)

    for l in range(1, NUM_LAYER):
        p = params['layers'][l]
        w1f, b1f, w2f, b2f = _fold(p['W1'], p['b1'], p['bn1_g'], p['bn1_b'],
                                   p['W2'], p['b2'], p['bn_g'], p['bn_b'])
        epsr = jnp.broadcast_to(1.0 + p['eps'], (1, EMB)).astype(jnp.float32)

        acc = _sc_agg(hl, eip)

        if l < NUM_LAYER - 1:
            q = params['vn_mlps'][l]
            q1f, qb1f, q2f, qb2f = _fold(q['W1'], q['b1'], q['bn1_g'],
                                         q['bn1_b'], q['W2'], q['b2'],
                                         q['bn2_g'], q['bn2_b'])
            vn_next = _tc_vn(pool, vn, q1f, qb1f, q2f, qb2f)
        else:
            vn_next = jnp.zeros((NUM_GRAPHS, EMB), jnp.float32)

        hl, pool = _tc_layer(hl, acc, b3, w1f, b1f, w2f, b2f, epsr, vn_next,
                             last=(l == NUM_LAYER - 1))
        vn = vn_next

    q = params['pred_W']
    return _tc_final(pool, counts, q, params['pred_b'][None, :])


# speedup vs baseline: 1.0294x; 1.0294x over previous
import functools

import jax
import jax.numpy as jnp
from jax import lax
from jax.experimental import pallas as pl
from jax.experimental.pallas import tpu as pltpu
from jax.experimental.pallas import tpu_sc as plsc

N = 10000
E = 320000
EMB = 128
NUM_LAYER = 5
NUM_GRAPHS = 64
NUM_CLASS = 10
BN_EPS = 1e-5

NC = 2
NS = 16
NW = NC * NS
CHUNK = 64
KB2 = 2 * (-(-E // (NW * CHUNK)))
KB0 = 238
KB1 = KB2 - KB0
TOTCH = NS * KB2
EP = TOTCH * CHUNK
ACC_ROWS = 10240
RPT = ACC_ROWS // NS
DUMMY_ROW = N + 8

BLK = 1000
GRID = N // BLK


NBUF = 4
DG = NBUF - 1
DI = 6
NIDX = 8


def _sc_agg_body(hl, eip, out, idx_v, rows_v, acc, sem_i, sem_g,
                 sem_s):
    c = lax.axis_index("c")
    s = lax.axis_index("s")
    kb = jnp.where(c == 0, KB0, KB1)
    base = jnp.where(c == 0, s * KB0, NS * KB0 + s * KB1)

    zv = jnp.zeros((16,), jnp.float32)

    def zstore(k, carry):
        r = lax.shift_right_logical(k, 3)
        q = lax.bitwise_and(k, 7)
        rows_v[0, r, pl.ds(q * 16, 16)] = zv
        return carry

    lax.fori_loop(0, CHUNK * (EMB // 16), zstore, 0, unroll=False)
    for t in range(RPT // CHUNK):
        pltpu.sync_copy(rows_v.at[0], acc.at[pl.ds(s * RPT + t * CHUNK,
                                                   CHUNK)])
    plsc.subcore_barrier()

    def fire_idx(j):
        sl = lax.rem(j, NIDX)
        pltpu.async_copy(eip.at[base + j], idx_v.at[sl], sem_i.at[sl])

    def wait_idx(j):
        sl = lax.rem(j, NIDX)
        pltpu.make_async_copy(eip.at[base + j], idx_v.at[sl],
                              sem_i.at[sl]).wait()

    def gather(j, b):
        sl = lax.rem(j, NIDX)
        pltpu.async_copy(hl.at[idx_v.at[sl, 0]], rows_v.at[b], sem_g.at[b])

    def wait_g(j, b):
        sl = lax.rem(j, NIDX)
        pltpu.make_async_copy(hl.at[idx_v.at[sl, 0]], rows_v.at[b],
                              sem_g.at[b]).wait()

    def scat(j, b):
        sl = lax.rem(j, NIDX)
        pltpu.async_copy(rows_v.at[b], acc.at[idx_v.at[sl, 1]], sem_s.at[b],
                         add=True)

    def wait_s(j, b):
        sl = lax.rem(j, NIDX)
        pltpu.make_async_copy(rows_v.at[b], acc.at[idx_v.at[sl, 1]],
                              sem_s.at[b]).wait()

    for k in range(DI):
        fire_idx(k)
    for k in range(DG):
        wait_idx(k)
        gather(k, k)

    wait_g(0, 0)
    scat(0, 0)
    wait_idx(DG)
    gather(DG, DG)
    fire_idx(DI)

    def body(fire, j, carry):
        bp = lax.rem(j - 1, NBUF)
        wait_s(j - 1, bp)
        wait_idx(j + DG)
        gather(j + DG, bp)
        if fire:
            fire_idx(j + DI)
        b = lax.rem(j, NBUF)
        wait_g(j, b)
        scat(j, b)
        return carry

    lax.fori_loop(1, kb - DI, functools.partial(body, True), 0, unroll=False)
    lax.fori_loop(kb - DI, kb - DG, functools.partial(body, False), 0,
                  unroll=False)

    for dj in range(DG):
        j = kb - DG + dj
        b = lax.rem(j, NBUF)
        wait_g(j, b)
        scat(j, b)
    for dt in range(DG + 1):
        t = kb - DG - 1 + dt
        wait_s(t, lax.rem(t, NBUF))

    plsc.subcore_barrier()
    pltpu.sync_copy(acc.at[pl.ds(s * RPT, RPT)], out.at[c, pl.ds(s * RPT, RPT)])


NI2 = 2


DPAD = 1024


DROWS = (GRID + 1) * DPAD
DSLC = DPAD


def _sc_deg_body(eip, zeros1, out, dbuf, tidx, ones_v, acc1, sem_i, sem_s):
    c = lax.axis_index("c")
    s = lax.axis_index("s")
    kb = jnp.where(c == 0, KB0, KB1)
    base = jnp.where(c == 0, s * KB0, NS * KB0 + s * KB1)

    @pl.when(s < GRID + 1)
    def _():
        pltpu.sync_copy(zeros1.at[s], acc1.at[pl.ds(s * DPAD, DPAD)])
    for q in range(CHUNK // 16):
        ones_v[pl.ds(q * 16, 16)] = jnp.ones((16,), jnp.float32)
    plsc.subcore_barrier()

    def fire(j):
        sl = lax.rem(j, NI2)
        pltpu.async_copy(eip.at[base + j], dbuf.at[sl], sem_i.at[sl])

    def wait(j):
        sl = lax.rem(j, NI2)
        pltpu.make_async_copy(eip.at[base + j], dbuf.at[sl],
                              sem_i.at[sl]).wait()

    def scat(j):
        sl = lax.rem(j, NI2)
        pltpu.async_copy(ones_v, acc1.at[tidx.at[sl]], sem_s.at[sl],
                         add=True)

    def wait_s(j):
        sl = lax.rem(j, NI2)
        pltpu.make_async_copy(ones_v, acc1.at[tidx.at[sl]],
                              sem_s.at[sl]).wait()

    def xform(j):
        sl = lax.rem(j, NI2)
        for q in range(CHUNK // 16):
            iv = dbuf[sl, 1, pl.ds(q * 16, 16)]
            gv = lax.shift_right_logical(iv * 33555, 25)
            fi = iv - gv * BLK + lax.shift_left(gv, 10)
            tidx[sl, pl.ds(q * 16, 16)] = fi

    fire(0)
    fire(1)
    wait(0)
    xform(0)
    fire(2)
    scat(0)
    wait(1)
    xform(1)
    fire(3)
    scat(1)

    def body(fires, j, carry):
        wait(j)
        wait_s(j - 2)
        xform(j)
        if fires:
            fire(j + NI2)
        scat(j)
        return carry

    lax.fori_loop(2, kb - NI2, functools.partial(body, True), 0,
                  unroll=False)
    lax.fori_loop(kb - NI2, kb, functools.partial(body, False), 0,
                  unroll=False)

    wait_s(kb - 2)
    wait_s(kb - 1)
    plsc.subcore_barrier()

    @pl.when(s < GRID)
    def _():
        pltpu.sync_copy(acc1.at[pl.ds(s * DPAD, DPAD)], out.at[s, c])


@functools.partial(jax.jit, static_argnames=())
def _sc_deg(eip, zeros1):
    mesh = plsc.VectorSubcoreMesh(core_axis_name="c", subcore_axis_name="s")
    f = pl.kernel(
        _sc_deg_body,
        out_type=jax.ShapeDtypeStruct((GRID, NC, DPAD), jnp.float32),
        mesh=mesh,
        scratch_types=[
            pltpu.VMEM((NI2, 2, CHUNK), jnp.int32),
            pltpu.VMEM((NI2, CHUNK), jnp.int32),
            pltpu.VMEM((CHUNK,), jnp.float32),
            pltpu.VMEM_SHARED((DROWS,), jnp.float32),
            pltpu.SemaphoreType.DMA((NI2,)),
            pltpu.SemaphoreType.DMA((NI2,)),
        ],
    )
    return f(eip, zeros1)


@functools.partial(jax.jit, static_argnames=())
def _sc_agg(hl, eip):
    mesh = plsc.VectorSubcoreMesh(core_axis_name="c", subcore_axis_name="s")
    f = pl.kernel(
        _sc_agg_body,
        out_type=jax.ShapeDtypeStruct((NC, ACC_ROWS, EMB), jnp.float32),
        mesh=mesh,
        scratch_types=[
            pltpu.VMEM((NIDX, 2, CHUNK), jnp.int32),
            pltpu.VMEM((NBUF, CHUNK, EMB), jnp.float32),
            pltpu.VMEM_SHARED((ACC_ROWS, EMB), jnp.float32),
            pltpu.SemaphoreType.DMA((NIDX,)),
            pltpu.SemaphoreType.DMA((NBUF,)),
            pltpu.SemaphoreType.DMA((NBUF,)),
        ],
    )
    return f(hl, eip)


def _one_hot_t(b3_blk):
    bt = b3_blk.reshape(1, BLK)
    gi = lax.broadcasted_iota(jnp.int32, (NUM_GRAPHS, BLK), 0)
    return (jnp.broadcast_to(bt, (NUM_GRAPHS, BLK)) == gi).astype(jnp.float32)


def _mlp_tail(i, last, z, b3_ref, w1_ref, b1_ref, w2_ref, b2_ref, vn_ref,
              hln_ref, pool_ref):
    y = jnp.dot(z, w1_ref[...], preferred_element_type=jnp.float32) + b1_ref[...]
    y = jnp.maximum(y, 0.0)
    h = jnp.dot(y, w2_ref[...], preferred_element_type=jnp.float32) + b2_ref[...]
    if not last:
        h = jnp.maximum(h, 0.0)
    bT = _one_hot_t(b3_ref[...])
    vnb = lax.dot_general(bT, vn_ref[...], (((0,), (0,)), ((), ())),
                          preferred_element_type=jnp.float32)
    hln = h + vnb
    hln_ref[...] = hln
    p = jnp.dot(bT, hln, preferred_element_type=jnp.float32)

    @pl.when(i == 0)
    def _():
        pool_ref[...] = p

    @pl.when(i > 0)
    def _():
        pool_ref[...] += p


def _tc_layer_body(last, hl_ref, acc_ref, b3_ref, w1_ref, b1_ref, w2_ref,
                   b2_ref, epsr_ref, vn_ref, hln_ref, pool_ref):
    i = pl.program_id(0)
    acc = acc_ref[...]
    z = hl_ref[...] * epsr_ref[...] + acc[0] + acc[1]
    _mlp_tail(i, last, z, b3_ref, w1_ref, b1_ref, w2_ref, b2_ref, vn_ref,
              hln_ref, pool_ref)


def _tc_layer0_body(deg_ref, b3_ref, c0_ref, epsr_ref, w1_ref, b1_ref,
                    w2_ref, b2_ref, vn_ref, hln_ref, pool_ref):
    i = pl.program_id(0)
    degT = jnp.sum(deg_ref[...].reshape(NC, DPAD), axis=0,
                   keepdims=True)[:, :BLK]
    c0 = c0_ref[...]
    z = (jnp.broadcast_to(c0 * epsr_ref[...], (BLK, EMB))
         + lax.dot_general(degT, c0, (((0,), (0,)), ((), ())),
                           preferred_element_type=jnp.float32))
    _mlp_tail(i, False, z, b3_ref, w1_ref, b1_ref, w2_ref, b2_ref, vn_ref,
              hln_ref, pool_ref)


def _tc_layer0(deg_p, b3, c0row, epsr, w1f, b1f, w2f, b2f, vn_next):
    return pl.pallas_call(
        _tc_layer0_body,
        grid=(GRID,),
        in_specs=[
            pl.BlockSpec((1, NC, DPAD), lambda i: (i, 0, 0)),
            pl.BlockSpec((1, 1, BLK), lambda i: (i, 0, 0)),
            pl.BlockSpec((1, EMB), lambda i: (0, 0)),
            pl.BlockSpec((1, EMB), lambda i: (0, 0)),
            pl.BlockSpec((EMB, 2 * EMB), lambda i: (0, 0)),
            pl.BlockSpec((1, 2 * EMB), lambda i: (0, 0)),
            pl.BlockSpec((2 * EMB, EMB), lambda i: (0, 0)),
            pl.BlockSpec((1, EMB), lambda i: (0, 0)),
            pl.BlockSpec((NUM_GRAPHS, EMB), lambda i: (0, 0)),
        ],
        out_specs=[
            pl.BlockSpec((BLK, EMB), lambda i: (i, 0)),
            pl.BlockSpec((NUM_GRAPHS, EMB), lambda i: (0, 0)),
        ],
        out_shape=[
            jax.ShapeDtypeStruct((N, EMB), jnp.float32),
            jax.ShapeDtypeStruct((NUM_GRAPHS, EMB), jnp.float32),
        ],
        compiler_params=pltpu.CompilerParams(
            dimension_semantics=("arbitrary",)),
    )(deg_p, b3, c0row, epsr, w1f, b1f, w2f, b2f, vn_next)


def _tc_layer(hl, acc, b3, w1f, b1f, w2f, b2f, epsr, vn_next, last):
    return pl.pallas_call(
        functools.partial(_tc_layer_body, last),
        grid=(GRID,),
        in_specs=[
            pl.BlockSpec((BLK, EMB), lambda i: (i, 0)),
            pl.BlockSpec((NC, BLK, EMB), lambda i: (0, i, 0)),
            pl.BlockSpec((1, 1, BLK), lambda i: (i, 0, 0)),
            pl.BlockSpec((EMB, 2 * EMB), lambda i: (0, 0)),
            pl.BlockSpec((1, 2 * EMB), lambda i: (0, 0)),
            pl.BlockSpec((2 * EMB, EMB), lambda i: (0, 0)),
            pl.BlockSpec((1, EMB), lambda i: (0, 0)),
            pl.BlockSpec((1, EMB), lambda i: (0, 0)),
            pl.BlockSpec((NUM_GRAPHS, EMB), lambda i: (0, 0)),
        ],
        out_specs=[
            pl.BlockSpec((BLK, EMB), lambda i: (i, 0)),
            pl.BlockSpec((NUM_GRAPHS, EMB), lambda i: (0, 0)),
        ],
        out_shape=[
            jax.ShapeDtypeStruct((N, EMB), jnp.float32),
            jax.ShapeDtypeStruct((NUM_GRAPHS, EMB), jnp.float32),
        ],
        compiler_params=pltpu.CompilerParams(
            dimension_semantics=("arbitrary",)),
    )(hl, acc, b3, w1f, b1f, w2f, b2f, epsr, vn_next)


def _tc_cnt_body(b3_ref, cnt_ref):
    i = pl.program_id(0)
    bT = _one_hot_t(b3_ref[...])
    c = jnp.broadcast_to(jnp.sum(bT, axis=1, keepdims=True),
                         (NUM_GRAPHS, EMB))

    @pl.when(i == 0)
    def _():
        cnt_ref[...] = c

    @pl.when(i > 0)
    def _():
        cnt_ref[...] += c


def _tc_cnt(b3):
    return pl.pallas_call(
        _tc_cnt_body,
        grid=(GRID,),
        in_specs=[pl.BlockSpec((1, 1, BLK), lambda i: (i, 0, 0))],
        out_specs=pl.BlockSpec((NUM_GRAPHS, EMB), lambda i: (0, 0)),
        out_shape=jax.ShapeDtypeStruct((NUM_GRAPHS, EMB), jnp.float32),
        compiler_params=pltpu.CompilerParams(
            dimension_semantics=("arbitrary",)),
    )(b3)


def _tc_vn0_body(cnt_ref, c0_ref, vn_ref, q1_ref, qb1_ref, q2_ref, qb2_ref,
                 out_ref):
    vt = (cnt_ref[...] * jnp.broadcast_to(c0_ref[...],
                                          (NUM_GRAPHS, EMB))
          + vn_ref[...])
    v = jnp.dot(vt, q1_ref[...], preferred_element_type=jnp.float32) + qb1_ref[...]
    v = jnp.maximum(v, 0.0)
    v = jnp.dot(v, q2_ref[...], preferred_element_type=jnp.float32) + qb2_ref[...]
    out_ref[...] = jnp.maximum(v, 0.0)


def _tc_vn0(cnt, c0row, vn, q1, qb1, q2, qb2):
    return pl.pallas_call(
        _tc_vn0_body,
        out_shape=jax.ShapeDtypeStruct((NUM_GRAPHS, EMB), jnp.float32),
    )(cnt, c0row, vn, q1, qb1, q2, qb2)


def _tc_vn_body(pool_ref, vn_ref, q1_ref, qb1_ref, q2_ref, qb2_ref, out_ref):
    vt = pool_ref[...] + vn_ref[...]
    v = jnp.dot(vt, q1_ref[...], preferred_element_type=jnp.float32) + qb1_ref[...]
    v = jnp.maximum(v, 0.0)
    v = jnp.dot(v, q2_ref[...], preferred_element_type=jnp.float32) + qb2_ref[...]
    out_ref[...] = jnp.maximum(v, 0.0)


def _tc_vn(pool, vn, q1, qb1, q2, qb2):
    return pl.pallas_call(
        _tc_vn_body,
        out_shape=jax.ShapeDtypeStruct((NUM_GRAPHS, EMB), jnp.float32),
    )(pool, vn, q1, qb1, q2, qb2)


def _tc_final_body(pool_ref, cnt_ref, w_ref, b_ref, out_ref):
    hg = pool_ref[...] / jnp.maximum(cnt_ref[...], 1.0)
    out_ref[...] = (jnp.dot(hg, w_ref[...], preferred_element_type=jnp.float32)
                    + b_ref[...])


def _tc_final(pool, cnt, w, b):
    return pl.pallas_call(
        _tc_final_body,
        out_shape=jax.ShapeDtypeStruct((NUM_GRAPHS, NUM_CLASS), jnp.float32),
    )(pool, cnt, w, b)


_BN_S = (1.0 + BN_EPS) ** -0.5


def _fold(W1, b1, g1, bb1, W2, b2, g2, bb2):
    s1 = _BN_S * g1
    s2 = _BN_S * g2
    return (W1 * s1[None, :], (b1 * s1 + bb1)[None, :],
            W2 * s2[None, :], (b2 * s2 + bb2)[None, :])


def kernel(x, edge_index, batch, params):
    del x
    src = edge_index[0]
    dst = edge_index[1]
    pad = EP - E
    srcp = jnp.concatenate([src, jnp.zeros((pad,), jnp.int32)]).reshape(
        TOTCH, 1, CHUNK)
    dstp = jnp.concatenate(
        [dst, jnp.full((pad,), DUMMY_ROW, jnp.int32)]).reshape(TOTCH, 1,
                                                               CHUNK)
    eip = jnp.concatenate([srcp, dstp], axis=1)
    b3 = batch.reshape(GRID, 1, BLK)

    c0row = (params['atom_emb'][0] + params['vn_emb'][0])[None, :]
    vn = jnp.broadcast_to(params['vn_emb'][0], (NUM_GRAPHS, EMB))

    zeros1 = jnp.zeros((GRID + 1, DPAD), jnp.float32)
    counts = _tc_cnt(b3)
    deg_p = _sc_deg(eip, zeros1)

    p = params['layers'][0]
    w1f, b1f, w2f, b2f = _fold(p['W1'], p['b1'], p['bn1_g'], p['bn1_b'],
                               p['W2'], p['b2'], p['bn_g'], p['bn_b'])
    epsr = jnp.broadcast_to(1.0 + p['eps'], (1, EMB)).astype(jnp.float32)
    q = params['vn_mlps'][0]
    q1f, qb1f, q2f, qb2f = _fold(q['W1'], q['b1'], q['bn1_g'], q['bn1_b'],
                                 q['W2'], q['b2'], q['bn2_g'], q['bn2_b'])
    vn = _tc_vn0(counts, c0row, vn, q1f, qb1f, q2f, qb2f)
    hl, pool = _tc_layer0(deg_p, b3, c0row, epsr, w1f, b1f, w2f, b2f, vn)

    for l in range(1, NUM_LAYER):
        p = params['layers'][l]
        w1f, b1f, w2f, b2f = _fold(p['W1'], p['b1'], p['bn1_g'], p['bn1_b'],
                                   p['W2'], p['b2'], p['bn_g'], p['bn_b'])
        epsr = jnp.broadcast_to(1.0 + p['eps'], (1, EMB)).astype(jnp.float32)

        acc = _sc_agg(hl, eip)

        if l < NUM_LAYER - 1:
            q = params['vn_mlps'][l]
            q1f, qb1f, q2f, qb2f = _fold(q['W1'], q['b1'], q['bn1_g'],
                                         q['bn1_b'], q['W2'], q['b2'],
                                         q['bn2_g'], q['bn2_b'])
            vn_next = _tc_vn(pool, vn, q1f, qb1f, q2f, qb2f)
        else:
            vn_next = jnp.zeros((NUM_GRAPHS, EMB), jnp.float32)

        hl, pool = _tc_layer(hl, acc, b3, w1f, b1f, w2f, b2f, epsr, vn_next,
                             last=(l == NUM_LAYER - 1))
        vn = vn_next

    q = params['pred_W']
    return _tc_final(pool, counts, q, params['pred_b'][None, :])


# speedup vs baseline: 1.0610x; 1.0307x over previous
import functools

import jax
import jax.numpy as jnp
from jax import lax
from jax.experimental import pallas as pl
from jax.experimental.pallas import tpu as pltpu
from jax.experimental.pallas import tpu_sc as plsc

N = 10000
E = 320000
EMB = 128
NUM_LAYER = 5
NUM_GRAPHS = 64
NUM_CLASS = 10
BN_EPS = 1e-5

NC = 2
NS = 16
NW = NC * NS
CHUNK = 64
KB2 = 2 * (-(-E // (NW * CHUNK)))
KB0 = 238
KB1 = KB2 - KB0
TOTCH = NS * KB2
EP = TOTCH * CHUNK
ACC_ROWS = 10240
RPT = ACC_ROWS // NS
DUMMY_ROW = N + 8

BLK = 1000
GRID = N // BLK


NBUF = 4
DG = NBUF - 1
DI = 6
NIDX = 8


def _sc_agg_body(hl, eip, out, idx_v, rows_v, acc, sem_i, sem_g,
                 sem_s):
    c = lax.axis_index("c")
    s = lax.axis_index("s")
    kb = jnp.where(c == 0, KB0, KB1)
    base = jnp.where(c == 0, s * KB0, NS * KB0 + s * KB1)

    zv = jnp.zeros((16,), jnp.float32)

    def zstore(k, carry):
        r = lax.shift_right_logical(k, 3)
        q = lax.bitwise_and(k, 7)
        rows_v[0, r, pl.ds(q * 16, 16)] = zv
        return carry

    lax.fori_loop(0, CHUNK * (EMB // 16), zstore, 0, unroll=False)
    for t in range(RPT // CHUNK):
        pltpu.sync_copy(rows_v.at[0], acc.at[pl.ds(s * RPT + t * CHUNK,
                                                   CHUNK)])
    plsc.subcore_barrier()

    def fire_idx(j):
        sl = lax.rem(j, NIDX)
        pltpu.async_copy(eip.at[base + j], idx_v.at[sl], sem_i.at[sl])

    def wait_idx(j):
        sl = lax.rem(j, NIDX)
        pltpu.make_async_copy(eip.at[base + j], idx_v.at[sl],
                              sem_i.at[sl]).wait()

    def gather(j, b):
        sl = lax.rem(j, NIDX)
        pltpu.async_copy(hl.at[idx_v.at[sl, 0]], rows_v.at[b], sem_g.at[b])

    def wait_g(j, b):
        sl = lax.rem(j, NIDX)
        pltpu.make_async_copy(hl.at[idx_v.at[sl, 0]], rows_v.at[b],
                              sem_g.at[b]).wait()

    def scat(j, b):
        sl = lax.rem(j, NIDX)
        pltpu.async_copy(rows_v.at[b], acc.at[idx_v.at[sl, 1]], sem_s.at[b],
                         add=True)

    def wait_s(j, b):
        sl = lax.rem(j, NIDX)
        pltpu.make_async_copy(rows_v.at[b], acc.at[idx_v.at[sl, 1]],
                              sem_s.at[b]).wait()

    for k in range(DI):
        fire_idx(k)
    for k in range(DG):
        wait_idx(k)
        gather(k, k)

    wait_g(0, 0)
    scat(0, 0)
    wait_idx(DG)
    gather(DG, DG)
    fire_idx(DI)

    def body(fire, j, carry):
        bp = lax.rem(j - 1, NBUF)
        wait_s(j - 1, bp)
        wait_idx(j + DG)
        gather(j + DG, bp)
        if fire:
            fire_idx(j + DI)
        b = lax.rem(j, NBUF)
        wait_g(j, b)
        scat(j, b)
        return carry

    lax.fori_loop(1, kb - DI, functools.partial(body, True), 0, unroll=False)
    lax.fori_loop(kb - DI, kb - DG, functools.partial(body, False), 0,
                  unroll=False)

    for dj in range(DG):
        j = kb - DG + dj
        b = lax.rem(j, NBUF)
        wait_g(j, b)
        scat(j, b)
    for dt in range(DG + 1):
        t = kb - DG - 1 + dt
        wait_s(t, lax.rem(t, NBUF))

    plsc.subcore_barrier()
    pltpu.sync_copy(acc.at[pl.ds(s * RPT, RPT)], out.at[c, pl.ds(s * RPT, RPT)])


NI2 = 2


DPAD = 1024


DROWS = (GRID + 1) * DPAD
DSLC = DPAD


def _sc_deg_body(eip, zeros1, out, dbuf, tidx, ones_v, acc1, sem_i, sem_s):
    c = lax.axis_index("c")
    s = lax.axis_index("s")
    kb = KB2 // 2
    base = (c * NS + s) * kb

    @pl.when(s < GRID + 1)
    def _():
        pltpu.sync_copy(zeros1.at[s], acc1.at[pl.ds(s * DPAD, DPAD)])
    for q in range(CHUNK // 16):
        ones_v[pl.ds(q * 16, 16)] = jnp.ones((16,), jnp.float32)
    plsc.subcore_barrier()

    def fire(j):
        sl = lax.rem(j, NI2)
        pltpu.async_copy(eip.at[base + j], dbuf.at[sl], sem_i.at[sl])

    def wait(j):
        sl = lax.rem(j, NI2)
        pltpu.make_async_copy(eip.at[base + j], dbuf.at[sl],
                              sem_i.at[sl]).wait()

    def scat(j):
        sl = lax.rem(j, NI2)
        pltpu.async_copy(ones_v, acc1.at[tidx.at[sl]], sem_s.at[sl],
                         add=True)

    def wait_s(j):
        sl = lax.rem(j, NI2)
        pltpu.make_async_copy(ones_v, acc1.at[tidx.at[sl]],
                              sem_s.at[sl]).wait()

    def xform(j):
        sl = lax.rem(j, NI2)
        for q in range(CHUNK // 16):
            iv = dbuf[sl, 1, pl.ds(q * 16, 16)]
            gv = lax.shift_right_logical(iv * 33555, 25)
            fi = iv - gv * BLK + lax.shift_left(gv, 10)
            tidx[sl, pl.ds(q * 16, 16)] = fi

    fire(0)
    fire(1)
    wait(0)
    xform(0)
    fire(2)
    scat(0)
    wait(1)
    xform(1)
    fire(3)
    scat(1)

    def body(fires, j, carry):
        wait(j)
        wait_s(j - 2)
        xform(j)
        if fires:
            fire(j + NI2)
        scat(j)
        return carry

    lax.fori_loop(2, kb - NI2, functools.partial(body, True), 0,
                  unroll=False)
    lax.fori_loop(kb - NI2, kb, functools.partial(body, False), 0,
                  unroll=False)

    wait_s(kb - 2)
    wait_s(kb - 1)
    plsc.subcore_barrier()

    @pl.when(s < GRID)
    def _():
        pltpu.sync_copy(acc1.at[pl.ds(s * DPAD, DPAD)], out.at[s, c])


@functools.partial(jax.jit, static_argnames=())
def _sc_deg(eip, zeros1):
    mesh = plsc.VectorSubcoreMesh(core_axis_name="c", subcore_axis_name="s")
    f = pl.kernel(
        _sc_deg_body,
        out_type=jax.ShapeDtypeStruct((GRID, NC, DPAD), jnp.float32),
        mesh=mesh,
        scratch_types=[
            pltpu.VMEM((NI2, 2, CHUNK), jnp.int32),
            pltpu.VMEM((NI2, CHUNK), jnp.int32),
            pltpu.VMEM((CHUNK,), jnp.float32),
            pltpu.VMEM_SHARED((DROWS,), jnp.float32),
            pltpu.SemaphoreType.DMA((NI2,)),
            pltpu.SemaphoreType.DMA((NI2,)),
        ],
    )
    return f(eip, zeros1)


@functools.partial(jax.jit, static_argnames=())
def _sc_agg(hl, eip):
    mesh = plsc.VectorSubcoreMesh(core_axis_name="c", subcore_axis_name="s")
    f = pl.kernel(
        _sc_agg_body,
        out_type=jax.ShapeDtypeStruct((NC, ACC_ROWS, EMB), jnp.float32),
        mesh=mesh,
        scratch_types=[
            pltpu.VMEM((NIDX, 2, CHUNK), jnp.int32),
            pltpu.VMEM((NBUF, CHUNK, EMB), jnp.float32),
            pltpu.VMEM_SHARED((ACC_ROWS, EMB), jnp.float32),
            pltpu.SemaphoreType.DMA((NIDX,)),
            pltpu.SemaphoreType.DMA((NBUF,)),
            pltpu.SemaphoreType.DMA((NBUF,)),
        ],
    )
    return f(hl, eip)


def _one_hot_t(b3_blk):
    bt = b3_blk.reshape(1, BLK)
    gi = lax.broadcasted_iota(jnp.int32, (NUM_GRAPHS, BLK), 0)
    return (jnp.broadcast_to(bt, (NUM_GRAPHS, BLK)) == gi).astype(jnp.float32)


def _mlp_tail(i, last, z, b3_ref, w1_ref, b1_ref, w2_ref, b2_ref, vn_ref,
              hln_ref, pool_ref):
    y = jnp.dot(z, w1_ref[...], preferred_element_type=jnp.float32) + b1_ref[...]
    y = jnp.maximum(y, 0.0)
    h = jnp.dot(y, w2_ref[...], preferred_element_type=jnp.float32) + b2_ref[...]
    if not last:
        h = jnp.maximum(h, 0.0)
    bT = _one_hot_t(b3_ref[...])
    vnb = lax.dot_general(bT, vn_ref[...], (((0,), (0,)), ((), ())),
                          preferred_element_type=jnp.float32)
    hln = h + vnb
    hln_ref[...] = hln
    p = jnp.dot(bT, hln, preferred_element_type=jnp.float32)

    @pl.when(i == 0)
    def _():
        pool_ref[...] = p

    @pl.when(i > 0)
    def _():
        pool_ref[...] += p


def _tc_layer_body(last, hl_ref, acc_ref, b3_ref, w1_ref, b1_ref, w2_ref,
                   b2_ref, epsr_ref, vn_ref, hln_ref, pool_ref):
    i = pl.program_id(0)
    acc = acc_ref[...]
    z = hl_ref[...] * epsr_ref[...] + acc[0] + acc[1]
    _mlp_tail(i, last, z, b3_ref, w1_ref, b1_ref, w2_ref, b2_ref, vn_ref,
              hln_ref, pool_ref)


def _tc_layer0_body(deg_ref, b3_ref, c0_ref, epsr_ref, w1_ref, b1_ref,
                    w2_ref, b2_ref, vn_ref, hln_ref, pool_ref):
    i = pl.program_id(0)
    degT = jnp.sum(deg_ref[...].reshape(NC, DPAD), axis=0,
                   keepdims=True)[:, :BLK]
    c0 = c0_ref[...]
    z = (jnp.broadcast_to(c0 * epsr_ref[...], (BLK, EMB))
         + lax.dot_general(degT, c0, (((0,), (0,)), ((), ())),
                           preferred_element_type=jnp.float32))
    _mlp_tail(i, False, z, b3_ref, w1_ref, b1_ref, w2_ref, b2_ref, vn_ref,
              hln_ref, pool_ref)


def _tc_layer0(deg_p, b3, c0row, epsr, w1f, b1f, w2f, b2f, vn_next):
    return pl.pallas_call(
        _tc_layer0_body,
        grid=(GRID,),
        in_specs=[
            pl.BlockSpec((1, NC, DPAD), lambda i: (i, 0, 0)),
            pl.BlockSpec((1, 1, BLK), lambda i: (i, 0, 0)),
            pl.BlockSpec((1, EMB), lambda i: (0, 0)),
            pl.BlockSpec((1, EMB), lambda i: (0, 0)),
            pl.BlockSpec((EMB, 2 * EMB), lambda i: (0, 0)),
            pl.BlockSpec((1, 2 * EMB), lambda i: (0, 0)),
            pl.BlockSpec((2 * EMB, EMB), lambda i: (0, 0)),
            pl.BlockSpec((1, EMB), lambda i: (0, 0)),
            pl.BlockSpec((NUM_GRAPHS, EMB), lambda i: (0, 0)),
        ],
        out_specs=[
            pl.BlockSpec((BLK, EMB), lambda i: (i, 0)),
            pl.BlockSpec((NUM_GRAPHS, EMB), lambda i: (0, 0)),
        ],
        out_shape=[
            jax.ShapeDtypeStruct((N, EMB), jnp.float32),
            jax.ShapeDtypeStruct((NUM_GRAPHS, EMB), jnp.float32),
        ],
        compiler_params=pltpu.CompilerParams(
            dimension_semantics=("arbitrary",)),
    )(deg_p, b3, c0row, epsr, w1f, b1f, w2f, b2f, vn_next)


def _tc_layer(hl, acc, b3, w1f, b1f, w2f, b2f, epsr, vn_next, last):
    return pl.pallas_call(
        functools.partial(_tc_layer_body, last),
        grid=(GRID,),
        in_specs=[
            pl.BlockSpec((BLK, EMB), lambda i: (i, 0)),
            pl.BlockSpec((NC, BLK, EMB), lambda i: (0, i, 0)),
            pl.BlockSpec((1, 1, BLK), lambda i: (i, 0, 0)),
            pl.BlockSpec((EMB, 2 * EMB), lambda i: (0, 0)),
            pl.BlockSpec((1, 2 * EMB), lambda i: (0, 0)),
            pl.BlockSpec((2 * EMB, EMB), lambda i: (0, 0)),
            pl.BlockSpec((1, EMB), lambda i: (0, 0)),
            pl.BlockSpec((1, EMB), lambda i: (0, 0)),
            pl.BlockSpec((NUM_GRAPHS, EMB), lambda i: (0, 0)),
        ],
        out_specs=[
            pl.BlockSpec((BLK, EMB), lambda i: (i, 0)),
            pl.BlockSpec((NUM_GRAPHS, EMB), lambda i: (0, 0)),
        ],
        out_shape=[
            jax.ShapeDtypeStruct((N, EMB), jnp.float32),
            jax.ShapeDtypeStruct((NUM_GRAPHS, EMB), jnp.float32),
        ],
        compiler_params=pltpu.CompilerParams(
            dimension_semantics=("arbitrary",)),
    )(hl, acc, b3, w1f, b1f, w2f, b2f, epsr, vn_next)


def _tc_cnt_body(b3_ref, cnt_ref):
    i = pl.program_id(0)
    bT = _one_hot_t(b3_ref[...])
    c = jnp.broadcast_to(jnp.sum(bT, axis=1, keepdims=True),
                         (NUM_GRAPHS, EMB))

    @pl.when(i == 0)
    def _():
        cnt_ref[...] = c

    @pl.when(i > 0)
    def _():
        cnt_ref[...] += c


def _tc_cnt(b3):
    return pl.pallas_call(
        _tc_cnt_body,
        grid=(GRID,),
        in_specs=[pl.BlockSpec((1, 1, BLK), lambda i: (i, 0, 0))],
        out_specs=pl.BlockSpec((NUM_GRAPHS, EMB), lambda i: (0, 0)),
        out_shape=jax.ShapeDtypeStruct((NUM_GRAPHS, EMB), jnp.float32),
        compiler_params=pltpu.CompilerParams(
            dimension_semantics=("arbitrary",)),
    )(b3)


def _tc_vn0_body(cnt_ref, c0_ref, vn_ref, q1_ref, qb1_ref, q2_ref, qb2_ref,
                 out_ref):
    vt = (cnt_ref[...] * jnp.broadcast_to(c0_ref[...],
                                          (NUM_GRAPHS, EMB))
          + vn_ref[...])
    v = jnp.dot(vt, q1_ref[...], preferred_element_type=jnp.float32) + qb1_ref[...]
    v = jnp.maximum(v, 0.0)
    v = jnp.dot(v, q2_ref[...], preferred_element_type=jnp.float32) + qb2_ref[...]
    out_ref[...] = jnp.maximum(v, 0.0)


def _tc_vn0(cnt, c0row, vn, q1, qb1, q2, qb2):
    return pl.pallas_call(
        _tc_vn0_body,
        out_shape=jax.ShapeDtypeStruct((NUM_GRAPHS, EMB), jnp.float32),
    )(cnt, c0row, vn, q1, qb1, q2, qb2)


def _tc_vn_body(pool_ref, vn_ref, q1_ref, qb1_ref, q2_ref, qb2_ref, out_ref):
    vt = pool_ref[...] + vn_ref[...]
    v = jnp.dot(vt, q1_ref[...], preferred_element_type=jnp.float32) + qb1_ref[...]
    v = jnp.maximum(v, 0.0)
    v = jnp.dot(v, q2_ref[...], preferred_element_type=jnp.float32) + qb2_ref[...]
    out_ref[...] = jnp.maximum(v, 0.0)


def _tc_vn(pool, vn, q1, qb1, q2, qb2):
    return pl.pallas_call(
        _tc_vn_body,
        out_shape=jax.ShapeDtypeStruct((NUM_GRAPHS, EMB), jnp.float32),
    )(pool, vn, q1, qb1, q2, qb2)


def _tc_final_body(pool_ref, cnt_ref, w_ref, b_ref, out_ref):
    hg = pool_ref[...] / jnp.maximum(cnt_ref[...], 1.0)
    out_ref[...] = (jnp.dot(hg, w_ref[...], preferred_element_type=jnp.float32)
                    + b_ref[...])


def _tc_final(pool, cnt, w, b):
    return pl.pallas_call(
        _tc_final_body,
        out_shape=jax.ShapeDtypeStruct((NUM_GRAPHS, NUM_CLASS), jnp.float32),
    )(pool, cnt, w, b)


_BN_S = (1.0 + BN_EPS) ** -0.5


def _fold(W1, b1, g1, bb1, W2, b2, g2, bb2):
    s1 = _BN_S * g1
    s2 = _BN_S * g2
    return (W1 * s1[None, :], (b1 * s1 + bb1)[None, :],
            W2 * s2[None, :], (b2 * s2 + bb2)[None, :])


def kernel(x, edge_index, batch, params):
    del x
    src = edge_index[0]
    dst = edge_index[1]
    pad = EP - E
    srcp = jnp.concatenate([src, jnp.zeros((pad,), jnp.int32)]).reshape(
        TOTCH, 1, CHUNK)
    dstp = jnp.concatenate(
        [dst, jnp.full((pad,), DUMMY_ROW, jnp.int32)]).reshape(TOTCH, 1,
                                                               CHUNK)
    eip = jnp.concatenate([srcp, dstp], axis=1)
    b3 = batch.reshape(GRID, 1, BLK)

    c0row = (params['atom_emb'][0] + params['vn_emb'][0])[None, :]
    vn = jnp.broadcast_to(params['vn_emb'][0], (NUM_GRAPHS, EMB))

    zeros1 = jnp.zeros((GRID + 1, DPAD), jnp.float32)
    counts = _tc_cnt(b3)
    deg_p = _sc_deg(eip, zeros1)

    p = params['layers'][0]
    w1f, b1f, w2f, b2f = _fold(p['W1'], p['b1'], p['bn1_g'], p['bn1_b'],
                               p['W2'], p['b2'], p['bn_g'], p['bn_b'])
    epsr = jnp.broadcast_to(1.0 + p['eps'], (1, EMB)).astype(jnp.float32)
    q = params['vn_mlps'][0]
    q1f, qb1f, q2f, qb2f = _fold(q['W1'], q['b1'], q['bn1_g'], q['bn1_b'],
                                 q['W2'], q['b2'], q['bn2_g'], q['bn2_b'])
    vn = _tc_vn0(counts, c0row, vn, q1f, qb1f, q2f, qb2f)
    hl, pool = _tc_layer0(deg_p, b3, c0row, epsr, w1f, b1f, w2f, b2f, vn)

    for l in range(1, NUM_LAYER):
        p = params['layers'][l]
        w1f, b1f, w2f, b2f = _fold(p['W1'], p['b1'], p['bn1_g'], p['bn1_b'],
                                   p['W2'], p['b2'], p['bn_g'], p['bn_b'])
        epsr = jnp.broadcast_to(1.0 + p['eps'], (1, EMB)).astype(jnp.float32)

        acc = _sc_agg(hl, eip)

        if l < NUM_LAYER - 1:
            q = params['vn_mlps'][l]
            q1f, qb1f, q2f, qb2f = _fold(q['W1'], q['b1'], q['bn1_g'],
                                         q['bn1_b'], q['W2'], q['b2'],
                                         q['bn2_g'], q['bn2_b'])
            vn_next = _tc_vn(pool, vn, q1f, qb1f, q2f, qb2f)
        else:
            vn_next = jnp.zeros((NUM_GRAPHS, EMB), jnp.float32)

        hl, pool = _tc_layer(hl, acc, b3, w1f, b1f, w2f, b2f, epsr, vn_next,
                             last=(l == NUM_LAYER - 1))
        vn = vn_next

    q = params['pred_W']
    return _tc_final(pool, counts, q, params['pred_b'][None, :])
